# baseline (speedup 1.0000x reference)
import jax
import jax.numpy as jnp
from jax.experimental import pallas as pl
from jax.experimental.pallas import tpu as pltpu

_EPS = 1e-12
_NBLK = 1024
_CHUNK = 512


def _netvlad_block(x_ref, w_ref, b_ref, mut_ref, o_ref,
                   acc_ref, x16_ref, a16_ref):
    n = pl.program_id(1)
    nb = pl.num_programs(1)
    f_dim = x_ref.shape[1]

    @pl.when(n == 0)
    def _():
        acc_ref[...] = jnp.zeros_like(acc_ref)
        x16_ref[f_dim:, :] = jnp.ones_like(x16_ref[f_dim:, :])

    x16_ref[:f_dim, :] = x_ref[0].astype(jnp.bfloat16)
    w = w_ref[...]
    bias = b_ref[...]
    for k in range(_NBLK // _CHUNK):
        xc = x16_ref[:f_dim, k * _CHUNK:(k + 1) * _CHUNK]
        logits = jax.lax.dot_general(
            xc, w, (((0,), (0,)), ((), ())),
            preferred_element_type=jnp.float32) + bias
        e = jnp.exp(logits)
        a = e / jnp.sum(e, axis=1, keepdims=True)
        a16_ref[k * _CHUNK:(k + 1) * _CHUNK, :] = a.astype(jnp.bfloat16)

    acc_ref[...] += jax.lax.dot_general(
        x16_ref[...], a16_ref[...], (((1,), (0,)), ((), ())),
        preferred_element_type=jnp.float32)

    @pl.when(n == nb - 1)
    def _():
        vlad = acc_ref[:f_dim, :] \
            - acc_ref[f_dim:f_dim + 1, :] * mut_ref[...]
        ssq = jnp.sum(vlad * vlad, axis=0, keepdims=True)
        vn = vlad / jnp.maximum(jnp.sqrt(ssq), _EPS)
        gss = jnp.sum(vn * vn, keepdims=True)
        out = vn / jnp.maximum(jnp.sqrt(gss), _EPS)
        o_ref[...] = out[None]


def kernel(x, W, b, mu):
    B, F, N = x.shape
    C = W.shape[1]
    out = pl.pallas_call(
        _netvlad_block,
        out_shape=jax.ShapeDtypeStruct((B, F, C), jnp.float32),
        grid=(B, N // _NBLK),
        in_specs=[
            pl.BlockSpec((1, F, _NBLK), lambda i, j: (i, 0, j)),
            pl.BlockSpec((F, C), lambda i, j: (0, 0)),
            pl.BlockSpec((1, C), lambda i, j: (0, 0)),
            pl.BlockSpec((F, C), lambda i, j: (0, 0)),
        ],
        out_specs=pl.BlockSpec((1, F, C), lambda i, j: (i, 0, 0)),
        scratch_shapes=[
            pltpu.VMEM((F + 16, C), jnp.float32),
            pltpu.VMEM((F + 16, _NBLK), jnp.bfloat16),
            pltpu.VMEM((_NBLK, C), jnp.bfloat16),
        ],
        compiler_params=pltpu.CompilerParams(
            dimension_semantics=("parallel", "arbitrary"),
        ),
        name="netvlad_fused",
    )(x, W.astype(jnp.bfloat16), b.reshape(1, C), mu.T)
    return out.swapaxes(1, 2).reshape(B, C * F)


# speedup vs baseline: 1.3431x; 1.3431x over previous
import jax
import jax.numpy as jnp
from jax.experimental import pallas as pl
from jax.experimental.pallas import tpu as pltpu

_EPS = 1e-12
_NBLK = 2048
_CHUNK = 512


def _netvlad_block(x_ref, w_ref, b_ref, mut_ref, o_ref,
                   acc_ref, x16_ref, a16_ref):
    n = pl.program_id(1)
    nb = pl.num_programs(1)
    f_dim = x_ref.shape[1]

    @pl.when(n == 0)
    def _():
        acc_ref[...] = jnp.zeros_like(acc_ref)
        x16_ref[f_dim:, :] = jnp.ones_like(x16_ref[f_dim:, :])

    x16_ref[:f_dim, :] = x_ref[0].astype(jnp.bfloat16)
    w = w_ref[...]
    bias = b_ref[...]
    for k in range(_NBLK // _CHUNK):
        xc = x16_ref[:f_dim, k * _CHUNK:(k + 1) * _CHUNK]
        logits = jax.lax.dot_general(
            xc, w, (((0,), (0,)), ((), ())),
            preferred_element_type=jnp.float32) + bias
        e = jnp.exp(logits)
        a = e / jnp.sum(e, axis=1, keepdims=True)
        a16_ref[k * _CHUNK:(k + 1) * _CHUNK, :] = a.astype(jnp.bfloat16)

    acc_ref[...] += jax.lax.dot_general(
        x16_ref[...], a16_ref[...], (((1,), (0,)), ((), ())),
        preferred_element_type=jnp.float32)

    @pl.when(n == nb - 1)
    def _():
        vlad = acc_ref[:f_dim, :] \
            - acc_ref[f_dim:f_dim + 1, :] * mut_ref[...]
        ssq = jnp.sum(vlad * vlad, axis=0, keepdims=True)
        vn = vlad / jnp.maximum(jnp.sqrt(ssq), _EPS)
        gss = jnp.sum(vn * vn, keepdims=True)
        out = vn / jnp.maximum(jnp.sqrt(gss), _EPS)
        o_ref[...] = out[None]


def kernel(x, W, b, mu):
    B, F, N = x.shape
    C = W.shape[1]
    out = pl.pallas_call(
        _netvlad_block,
        out_shape=jax.ShapeDtypeStruct((B, F, C), jnp.float32),
        grid=(B, N // _NBLK),
        in_specs=[
            pl.BlockSpec((1, F, _NBLK), lambda i, j: (i, 0, j)),
            pl.BlockSpec((F, C), lambda i, j: (0, 0)),
            pl.BlockSpec((1, C), lambda i, j: (0, 0)),
            pl.BlockSpec((F, C), lambda i, j: (0, 0)),
        ],
        out_specs=pl.BlockSpec((1, F, C), lambda i, j: (i, 0, 0)),
        scratch_shapes=[
            pltpu.VMEM((F + 16, C), jnp.float32),
            pltpu.VMEM((F + 16, _NBLK), jnp.bfloat16),
            pltpu.VMEM((_NBLK, C), jnp.bfloat16),
        ],
        compiler_params=pltpu.CompilerParams(
            dimension_semantics=("parallel", "arbitrary"),
        ),
        name="netvlad_fused",
    )(x, W.astype(jnp.bfloat16), b.reshape(1, C), mu.T)
    return out.swapaxes(1, 2).reshape(B, C * F)


# speedup vs baseline: 1.6899x; 1.2582x over previous
import jax
import jax.numpy as jnp
from jax.experimental import pallas as pl
from jax.experimental.pallas import tpu as pltpu

_EPS = 1e-12
_NBLK = 4096
_CHUNK = 512


def _netvlad_block(x_ref, w_ref, b_ref, mut_ref, o_ref,
                   acc_ref, x16_ref, a16_ref):
    n = pl.program_id(1)
    nb = pl.num_programs(1)
    f_dim = x_ref.shape[1]

    @pl.when(n == 0)
    def _():
        acc_ref[...] = jnp.zeros_like(acc_ref)
        x16_ref[f_dim:, :] = jnp.ones_like(x16_ref[f_dim:, :])

    x16_ref[:f_dim, :] = x_ref[0].astype(jnp.bfloat16)
    w = w_ref[...]
    bias = b_ref[...]
    for k in range(_NBLK // _CHUNK):
        xc = x16_ref[:f_dim, k * _CHUNK:(k + 1) * _CHUNK]
        logits = jax.lax.dot_general(
            xc, w, (((0,), (0,)), ((), ())),
            preferred_element_type=jnp.float32) + bias
        e = jnp.exp(logits)
        a = e / jnp.sum(e, axis=1, keepdims=True)
        a16_ref[k * _CHUNK:(k + 1) * _CHUNK, :] = a.astype(jnp.bfloat16)

    acc_ref[...] += jax.lax.dot_general(
        x16_ref[...], a16_ref[...], (((1,), (0,)), ((), ())),
        preferred_element_type=jnp.float32)

    @pl.when(n == nb - 1)
    def _():
        vlad = acc_ref[:f_dim, :] \
            - acc_ref[f_dim:f_dim + 1, :] * mut_ref[...]
        ssq = jnp.sum(vlad * vlad, axis=0, keepdims=True)
        vn = vlad / jnp.maximum(jnp.sqrt(ssq), _EPS)
        gss = jnp.sum(vn * vn, keepdims=True)
        out = vn / jnp.maximum(jnp.sqrt(gss), _EPS)
        o_ref[...] = out[None]


def kernel(x, W, b, mu):
    B, F, N = x.shape
    C = W.shape[1]
    out = pl.pallas_call(
        _netvlad_block,
        out_shape=jax.ShapeDtypeStruct((B, F, C), jnp.float32),
        grid=(B, N // _NBLK),
        in_specs=[
            pl.BlockSpec((1, F, _NBLK), lambda i, j: (i, 0, j)),
            pl.BlockSpec((F, C), lambda i, j: (0, 0)),
            pl.BlockSpec((1, C), lambda i, j: (0, 0)),
            pl.BlockSpec((F, C), lambda i, j: (0, 0)),
        ],
        out_specs=pl.BlockSpec((1, F, C), lambda i, j: (i, 0, 0)),
        scratch_shapes=[
            pltpu.VMEM((F + 16, C), jnp.float32),
            pltpu.VMEM((F + 16, _NBLK), jnp.bfloat16),
            pltpu.VMEM((_NBLK, C), jnp.bfloat16),
        ],
        compiler_params=pltpu.CompilerParams(
            dimension_semantics=("parallel", "arbitrary"),
        ),
        name="netvlad_fused",
    )(x, W.astype(jnp.bfloat16), b.reshape(1, C), mu.T)
    return out.swapaxes(1, 2).reshape(B, C * F)
